# Initial kernel scaffold; baseline (speedup 1.0000x reference)
import jax
import jax.numpy as jnp
from jax.experimental import pallas as pl


def kernel(bbox_pred, conf_pred, anchors, target_boxes, target_labels, conf_weight, bbox_weight):
    raise NotImplementedError("write your pallas kernel here")


# speedup vs baseline: 8.2876x; 8.2876x over previous
import functools

import jax
import jax.numpy as jnp
from jax.experimental import pallas as pl
from jax.experimental.pallas import tpu as pltpu

_R = 160
_L = 128
_NP = _R * _L
_BISECT_ITERS = 40


def _loss_kernel(conf_ref, bbox_ref, anc_ref, boxes_ref, labels_ref,
                 conf_out, bbox_out, *, n_anchors, n_targets, n_classes):
    row = jax.lax.broadcasted_iota(jnp.int32, (_R, _L), 0)
    col = jax.lax.broadcasted_iota(jnp.int32, (_R, _L), 1)
    flat = row * _L + col
    valid = flat < n_anchors

    ax1 = anc_ref[0]
    ay1 = anc_ref[1]
    ax2 = anc_ref[2]
    ay2 = anc_ref[3]
    area_a = (ax2 - ax1) * (ay2 - ay1)

    best_iou = jnp.full((_R, _L), -1.0, dtype=jnp.float32)
    best_t = jnp.zeros((_R, _L), dtype=jnp.int32)
    iou0 = None
    for t in range(n_targets):
        bx1 = boxes_ref[0, t, 0]
        by1 = boxes_ref[0, t, 1]
        bx2 = boxes_ref[0, t, 2]
        by2 = boxes_ref[0, t, 3]
        ix1 = jnp.maximum(ax1, bx1)
        iy1 = jnp.maximum(ay1, by1)
        ix2 = jnp.minimum(ax2, bx2)
        iy2 = jnp.minimum(ay2, by2)
        inter = jnp.maximum(ix2 - ix1, 0.0) * jnp.maximum(iy2 - iy1, 0.0)
        area_b = (bx2 - bx1) * (by2 - by1)
        union = area_a + area_b - inter
        iou = inter / (union + 1e-6)
        if t == 0:
            iou0 = iou
            best_iou = iou
        else:
            upd = iou > best_iou
            best_iou = jnp.where(upd, iou, best_iou)
            best_t = jnp.where(upd, t, best_t)

    pos = jnp.logical_and(best_iou >= 0.5, valid)
    neg = jnp.logical_and(best_iou < 0.4, valid)

    no_pos = jnp.logical_not(jnp.any(pos))
    iou0m = jnp.where(valid, iou0, -1.0)
    m0 = jnp.max(iou0m)
    cand = jnp.where(iou0m == m0, flat, _NP)
    bidx = jnp.min(cand)
    force = jnp.logical_and(no_pos, flat == bidx)
    pos = jnp.logical_or(pos, force)
    neg = jnp.logical_and(neg, jnp.logical_not(force))
    matched = jnp.where(force, 0, best_t)

    num_pos = jnp.sum(pos.astype(jnp.float32))

    matched_label = jnp.zeros((_R, _L), dtype=jnp.int32)
    for t in range(n_targets):
        matched_label = jnp.where(matched == t, labels_ref[0, 0, t], matched_label)
    anchor_label = jnp.where(pos, matched_label, 0)

    m = conf_ref[0, 0]
    for c in range(1, n_classes):
        m = jnp.maximum(m, conf_ref[0, c])
    s = jnp.zeros((_R, _L), dtype=jnp.float32)
    picked = jnp.zeros((_R, _L), dtype=jnp.float32)
    for c in range(n_classes):
        logit = conf_ref[0, c]
        s = s + jnp.exp(logit - m)
        picked = jnp.where(anchor_label == c, logit, picked)
    ce = m + jnp.log(s) - picked

    n_neg = jnp.sum(neg.astype(jnp.float32))
    k = jnp.minimum(3.0 * num_pos, n_neg)

    ce_neg = jnp.where(neg, ce, 0.0)
    hi0 = jnp.max(ce_neg)

    def bisect(_, carry):
        lo, hi = carry
        mid = 0.5 * (lo + hi)
        cnt = jnp.sum(jnp.where(jnp.logical_and(neg, ce > mid), 1.0, 0.0))
        take = cnt >= k
        return jnp.where(take, mid, lo), jnp.where(take, hi, mid)

    lo, hi = jax.lax.fori_loop(0, _BISECT_ITERS, bisect,
                               (jnp.float32(0.0), hi0))
    thr = lo
    sel = jnp.logical_and(neg, ce > thr)
    cnt_gt = jnp.sum(sel.astype(jnp.float32))
    sum_gt = jnp.sum(jnp.where(sel, ce, 0.0))
    topk = sum_gt + (k - cnt_gt) * thr

    pos_ce = jnp.sum(jnp.where(pos, ce, 0.0))
    conf_loss = (pos_ce + topk) / (num_pos + k)

    sl1 = jnp.zeros((_R, _L), dtype=jnp.float32)
    for j in range(4):
        mb = jnp.zeros((_R, _L), dtype=jnp.float32)
        for t in range(n_targets):
            mb = jnp.where(matched == t, boxes_ref[0, t, j], mb)
        d = bbox_ref[0, j] - mb
        ad = jnp.abs(d)
        sl1 = sl1 + jnp.where(ad < 1.0, 0.5 * ad * ad, ad - 0.5)
    bbox_loss = jnp.sum(jnp.where(pos, sl1, 0.0)) / num_pos

    conf_out[0, 0, 0] = conf_loss
    bbox_out[0, 0, 0] = bbox_loss


def kernel(bbox_pred, conf_pred, anchors, target_boxes, target_labels,
           conf_weight=1.0, bbox_weight=1.0):
    B, N, C = conf_pred.shape
    T = target_boxes.shape[1]
    padn = _NP - N

    anc = jnp.pad(anchors, ((0, padn), (0, 0))).T.reshape(4, _R, _L)
    bbox_t = jnp.pad(bbox_pred, ((0, 0), (0, padn), (0, 0))) \
        .transpose(0, 2, 1).reshape(B, 4, _R, _L)
    conf_t = jnp.pad(conf_pred, ((0, 0), (0, padn), (0, 0))) \
        .transpose(0, 2, 1).reshape(B, C, _R, _L)
    labels = target_labels.astype(jnp.int32).reshape(B, 1, T)

    body = functools.partial(_loss_kernel, n_anchors=N, n_targets=T,
                             n_classes=C)
    conf_out, bbox_out = pl.pallas_call(
        body,
        grid=(B,),
        in_specs=[
            pl.BlockSpec((1, C, _R, _L), lambda i: (i, 0, 0, 0)),
            pl.BlockSpec((1, 4, _R, _L), lambda i: (i, 0, 0, 0)),
            pl.BlockSpec((4, _R, _L), lambda i: (0, 0, 0)),
            pl.BlockSpec((1, T, 4), lambda i: (i, 0, 0),
                         memory_space=pltpu.SMEM),
            pl.BlockSpec((1, 1, T), lambda i: (i, 0, 0),
                         memory_space=pltpu.SMEM),
        ],
        out_specs=[
            pl.BlockSpec((1, 1, 1), lambda i: (i, 0, 0),
                         memory_space=pltpu.SMEM),
            pl.BlockSpec((1, 1, 1), lambda i: (i, 0, 0),
                         memory_space=pltpu.SMEM),
        ],
        out_shape=[
            jax.ShapeDtypeStruct((B, 1, 1), jnp.float32),
            jax.ShapeDtypeStruct((B, 1, 1), jnp.float32),
        ],
    )(conf_t, bbox_t, anc, target_boxes, labels)

    conf_loss = jnp.sum(conf_out) / B
    bbox_loss = jnp.sum(bbox_out) / B
    total = conf_weight * conf_loss + bbox_weight * bbox_loss
    return total, conf_loss, bbox_loss


# speedup vs baseline: 12.0808x; 1.4577x over previous
import functools

import jax
import jax.numpy as jnp
from jax.experimental import pallas as pl
from jax.experimental.pallas import tpu as pltpu

_R = 160
_L = 128
_NP = _R * _L
_BISECT_ITERS = 24


def _rsum(x):
    return jnp.sum(x, axis=(1, 2), keepdims=True)


def _loss_kernel(conf_ref, bbox_ref, anc_ref, boxes_ref, labels_ref,
                 conf_out, bbox_out, *, n_batch, n_anchors, n_targets,
                 n_classes):
    row = jax.lax.broadcasted_iota(jnp.int32, (1, _R, _L), 1)
    col = jax.lax.broadcasted_iota(jnp.int32, (1, _R, _L), 2)
    flat = row * _L + col
    valid = flat < n_anchors

    ax1 = anc_ref[0][None]
    ay1 = anc_ref[1][None]
    ax2 = anc_ref[2][None]
    ay2 = anc_ref[3][None]
    area_a = (ax2 - ax1) * (ay2 - ay1)

    best_iou = jnp.full((n_batch, _R, _L), -1.0, dtype=jnp.float32)
    best_t = jnp.zeros((n_batch, _R, _L), dtype=jnp.int32)
    iou0 = None
    for t in range(n_targets):
        bx1 = boxes_ref[t, 0].reshape(n_batch, 1, 1)
        by1 = boxes_ref[t, 1].reshape(n_batch, 1, 1)
        bx2 = boxes_ref[t, 2].reshape(n_batch, 1, 1)
        by2 = boxes_ref[t, 3].reshape(n_batch, 1, 1)
        ix1 = jnp.maximum(ax1, bx1)
        iy1 = jnp.maximum(ay1, by1)
        ix2 = jnp.minimum(ax2, bx2)
        iy2 = jnp.minimum(ay2, by2)
        inter = jnp.maximum(ix2 - ix1, 0.0) * jnp.maximum(iy2 - iy1, 0.0)
        area_b = (bx2 - bx1) * (by2 - by1)
        union = area_a + area_b - inter
        iou = inter / (union + 1e-6)
        if t == 0:
            iou0 = iou
            best_iou = iou
        else:
            upd = iou > best_iou
            best_iou = jnp.where(upd, iou, best_iou)
            best_t = jnp.where(upd, t, best_t)

    pos = jnp.logical_and(best_iou >= 0.5, valid)
    neg = jnp.logical_and(best_iou < 0.4, valid)

    no_pos = jnp.logical_not(jnp.any(pos, axis=(1, 2), keepdims=True))
    iou0m = jnp.where(valid, iou0, -1.0)
    m0 = jnp.max(iou0m, axis=(1, 2), keepdims=True)
    cand = jnp.where(iou0m == m0, flat, _NP)
    bidx = jnp.min(cand, axis=(1, 2), keepdims=True)
    force = jnp.logical_and(no_pos, flat == bidx)
    pos = jnp.logical_or(pos, force)
    neg = jnp.logical_and(neg, jnp.logical_not(force))
    matched = jnp.where(force, 0, best_t)

    num_pos = _rsum(pos.astype(jnp.float32))

    matched_label = jnp.zeros((n_batch, _R, _L), dtype=jnp.int32)
    for t in range(n_targets):
        lab_t = labels_ref[t].reshape(n_batch, 1, 1)
        matched_label = jnp.where(matched == t, lab_t, matched_label)
    anchor_label = jnp.where(pos, matched_label, 0)

    m = conf_ref[:, 0]
    for c in range(1, n_classes):
        m = jnp.maximum(m, conf_ref[:, c])
    s = jnp.zeros((n_batch, _R, _L), dtype=jnp.float32)
    picked = jnp.zeros((n_batch, _R, _L), dtype=jnp.float32)
    for c in range(n_classes):
        logit = conf_ref[:, c]
        s = s + jnp.exp(logit - m)
        picked = jnp.where(anchor_label == c, logit, picked)
    ce = m + jnp.log(s) - picked

    n_neg = _rsum(neg.astype(jnp.float32))
    k = jnp.minimum(3.0 * num_pos, n_neg)

    ce_neg = jnp.where(neg, ce, -1.0)
    hi0 = jnp.maximum(jnp.max(ce_neg, axis=(1, 2), keepdims=True), 0.0)

    def bisect(_, carry):
        lo, hi = carry
        mid = 0.5 * (lo + hi)
        cnt = _rsum((ce_neg > mid).astype(jnp.float32))
        take = cnt >= k
        return jnp.where(take, mid, lo), jnp.where(take, hi, mid)

    lo, hi = jax.lax.fori_loop(
        0, _BISECT_ITERS, bisect, (jnp.zeros_like(hi0), hi0))
    thr = lo
    sel = ce_neg > thr
    cnt_gt = _rsum(sel.astype(jnp.float32))
    sum_gt = _rsum(jnp.where(sel, ce_neg, 0.0))
    topk = sum_gt + (k - cnt_gt) * thr

    pos_ce = _rsum(jnp.where(pos, ce, 0.0))
    conf_loss = (pos_ce + topk) / (num_pos + k)

    sl1 = jnp.zeros((n_batch, _R, _L), dtype=jnp.float32)
    for j in range(4):
        mb = jnp.zeros((n_batch, _R, _L), dtype=jnp.float32)
        for t in range(n_targets):
            mb = jnp.where(matched == t, boxes_ref[t, j].reshape(n_batch, 1, 1), mb)
        d = bbox_ref[:, j] - mb
        ad = jnp.abs(d)
        sl1 = sl1 + jnp.where(ad < 1.0, 0.5 * ad * ad, ad - 0.5)
    bbox_loss = _rsum(jnp.where(pos, sl1, 0.0)) / num_pos

    conf_out[...] = conf_loss
    bbox_out[...] = bbox_loss


def kernel(bbox_pred, conf_pred, anchors, target_boxes, target_labels,
           conf_weight=1.0, bbox_weight=1.0):
    B, N, C = conf_pred.shape
    T = target_boxes.shape[1]
    padn = _NP - N

    anc = jnp.pad(anchors, ((0, padn), (0, 0))).T.reshape(4, _R, _L)
    bbox_t = jnp.pad(bbox_pred, ((0, 0), (0, padn), (0, 0))) \
        .transpose(0, 2, 1).reshape(B, 4, _R, _L)
    conf_t = jnp.pad(conf_pred, ((0, 0), (0, padn), (0, 0))) \
        .transpose(0, 2, 1).reshape(B, C, _R, _L)
    boxes_v = target_boxes.transpose(1, 2, 0).reshape(T, 4, B, 1)
    labels_v = target_labels.astype(jnp.int32).T.reshape(T, B, 1)

    body = functools.partial(_loss_kernel, n_batch=B, n_anchors=N,
                             n_targets=T, n_classes=C)
    conf_out, bbox_out = pl.pallas_call(
        body,
        out_shape=[
            jax.ShapeDtypeStruct((B, 1, 1), jnp.float32),
            jax.ShapeDtypeStruct((B, 1, 1), jnp.float32),
        ],
    )(conf_t, bbox_t, anc, boxes_v, labels_v)

    conf_loss = jnp.sum(conf_out) / B
    bbox_loss = jnp.sum(bbox_out) / B
    total = conf_weight * conf_loss + bbox_weight * bbox_loss
    return total, conf_loss, bbox_loss


# speedup vs baseline: 21.8024x; 1.8047x over previous
import functools

import jax
import jax.numpy as jnp
from jax.experimental import pallas as pl
from jax.experimental.pallas import tpu as pltpu

_BISECT_ITERS = 20


def _rsum(x):
    return jnp.sum(x, axis=1, keepdims=True)


def _loss_kernel(conf_ref, bbox_ref, anc_ref, boxes_ref, labels_ref,
                 conf_out, bbox_out, *, n_batch, n_anchors, n_targets,
                 n_classes):
    lane = jax.lax.broadcasted_iota(jnp.int32, (1, n_anchors), 1)

    anc = anc_ref[...]
    ax1 = anc[0]
    ay1 = anc[1]
    ax2 = anc[2]
    ay2 = anc[3]
    area_a = (ax2 - ax1) * (ay2 - ay1) + 1e-6

    best_iou = None
    best_lab = None
    mb = [None] * 4
    iou0 = None
    for t in range(n_targets):
        bx1 = boxes_ref[t, 0]
        by1 = boxes_ref[t, 1]
        bx2 = boxes_ref[t, 2]
        by2 = boxes_ref[t, 3]
        lab_t = labels_ref[t]
        ix1 = jnp.maximum(ax1, bx1)
        iy1 = jnp.maximum(ay1, by1)
        ix2 = jnp.minimum(ax2, bx2)
        iy2 = jnp.minimum(ay2, by2)
        inter = jnp.maximum(ix2 - ix1, 0.0) * jnp.maximum(iy2 - iy1, 0.0)
        area_b = (bx2 - bx1) * (by2 - by1)
        union = area_a + (area_b - inter)
        iou = inter / union
        if t == 0:
            iou0 = iou
            best_iou = iou
            best_lab = jnp.broadcast_to(lab_t, (n_batch, n_anchors))
            mb = [jnp.broadcast_to(b, (n_batch, n_anchors))
                  for b in (bx1, by1, bx2, by2)]
        else:
            upd = iou > best_iou
            best_iou = jnp.where(upd, iou, best_iou)
            best_lab = jnp.where(upd, lab_t, best_lab)
            mb = [jnp.where(upd, b, o)
                  for b, o in zip((bx1, by1, bx2, by2), mb)]

    pos = best_iou >= 0.5
    neg = best_iou < 0.4

    no_pos = jnp.logical_not(jnp.any(pos, axis=1, keepdims=True))
    m0 = jnp.max(iou0, axis=1, keepdims=True)
    cand = jnp.where(iou0 == m0, lane, n_anchors)
    bidx = jnp.min(cand, axis=1, keepdims=True)
    force = jnp.logical_and(no_pos, lane == bidx)
    pos = jnp.logical_or(pos, force)
    neg = jnp.logical_and(neg, jnp.logical_not(force))
    if n_targets > 1:
        best_lab = jnp.where(force, labels_ref[0], best_lab)
        mb = [jnp.where(force, boxes_ref[0, j], o)
              for j, o in enumerate(mb)]

    num_pos = _rsum(pos.astype(jnp.float32))
    anchor_label = jnp.where(pos, best_lab, 0)

    m = conf_ref[0]
    for c in range(1, n_classes):
        m = jnp.maximum(m, conf_ref[c])
    s = jnp.zeros((n_batch, n_anchors), dtype=jnp.float32)
    picked = jnp.zeros((n_batch, n_anchors), dtype=jnp.float32)
    for c in range(n_classes):
        logit = conf_ref[c]
        s = s + jnp.exp(logit - m)
        picked = jnp.where(anchor_label == c, logit, picked)
    ce = m + jnp.log(s) - picked

    n_neg = _rsum(neg.astype(jnp.float32))
    k = jnp.minimum(3.0 * num_pos, n_neg)

    ce_neg = jnp.where(neg, ce, -1.0)
    hi0 = jnp.maximum(jnp.max(ce_neg, axis=1, keepdims=True), 0.0)

    def bisect(_, carry):
        lo, hi = carry
        mid = 0.5 * (lo + hi)
        cnt = _rsum((ce_neg > mid).astype(jnp.float32))
        take = cnt >= k
        return jnp.where(take, mid, lo), jnp.where(take, hi, mid)

    lo, hi = jax.lax.fori_loop(
        0, _BISECT_ITERS, bisect, (jnp.zeros_like(hi0), hi0))
    thr = lo
    sel = ce_neg > thr
    cnt_gt = _rsum(sel.astype(jnp.float32))
    sum_gt = _rsum(jnp.where(sel, ce_neg, 0.0))
    topk = sum_gt + (k - cnt_gt) * thr

    pos_ce = _rsum(jnp.where(pos, ce, 0.0))
    conf_loss = (pos_ce + topk) / (num_pos + k)

    sl1 = jnp.zeros((n_batch, n_anchors), dtype=jnp.float32)
    for j in range(4):
        d = bbox_ref[j] - mb[j]
        ad = jnp.abs(d)
        sl1 = sl1 + jnp.where(ad < 1.0, 0.5 * ad * ad, ad - 0.5)
    bbox_loss = _rsum(jnp.where(pos, sl1, 0.0)) / num_pos

    conf_out[...] = conf_loss
    bbox_out[...] = bbox_loss


def kernel(bbox_pred, conf_pred, anchors, target_boxes, target_labels,
           conf_weight=1.0, bbox_weight=1.0):
    B, N, C = conf_pred.shape
    T = target_boxes.shape[1]

    anc = anchors.T.reshape(4, 1, N)
    bbox_t = bbox_pred.transpose(2, 0, 1)
    conf_t = conf_pred.transpose(2, 0, 1)
    boxes_v = target_boxes.transpose(1, 2, 0).reshape(T, 4, B, 1)
    labels_v = target_labels.astype(jnp.int32).T.reshape(T, B, 1)

    body = functools.partial(_loss_kernel, n_batch=B, n_anchors=N,
                             n_targets=T, n_classes=C)
    conf_out, bbox_out = pl.pallas_call(
        body,
        out_shape=[
            jax.ShapeDtypeStruct((B, 1), jnp.float32),
            jax.ShapeDtypeStruct((B, 1), jnp.float32),
        ],
    )(conf_t, bbox_t, anc, boxes_v, labels_v)

    conf_loss = jnp.sum(conf_out) / B
    bbox_loss = jnp.sum(bbox_out) / B
    total = conf_weight * conf_loss + bbox_weight * bbox_loss
    return total, conf_loss, bbox_loss
